# baseline (speedup 1.0000x reference)
import functools

import jax
import jax.numpy as jnp
from jax import lax
from jax.experimental import pallas as pl
from jax.experimental.pallas import tpu as pltpu
from jax.experimental.pallas import tpu_sc as plsc

_NUM_CODES = 8192
_CODE_DIM = 256
_COMMIT = 0.25

_ROWS = 8192
_BR = 512
_BC = 1024
_NI = _ROWS // _BR
_NJ = _NUM_CODES // _BC

_NW = 32
_RPW = _ROWS // _NW
_IDX_ROWS = _RPW // 128


def _dist_argmin_body(z_ref, e_ref, idx_ref, loss_ref, min_s, idx_s):
    i = pl.program_id(0)
    j = pl.program_id(1)
    zb = z_ref[...]
    eb = e_ref[...]
    mm = lax.dot_general(zb.astype(jnp.bfloat16), eb.astype(jnp.bfloat16),
                         (((1,), (1,)), ((), ())),
                         preferred_element_type=jnp.float32)
    zn = jnp.sum(zb * zb, axis=1, keepdims=True)
    en = jnp.sum(eb * eb, axis=1)[None, :]
    dist = (zn - 2.0 * mm) + en
    lmin = jnp.min(dist, axis=1, keepdims=True)
    iota = lax.broadcasted_iota(jnp.int32, dist.shape, 1)
    masked = jnp.where(dist == lmin, iota, jnp.int32(2**31 - 1))
    larg = jnp.min(masked, axis=1, keepdims=True) + j * _BC

    def _carry_round(x, jj):
        cond = (jj == 1) | (jj == 3) | (jj == 5)
        return jnp.where(cond, x.astype(jnp.bfloat16).astype(jnp.float32), x)

    @pl.when(j == 0)
    def _():
        min_s[...] = lmin
        idx_s[...] = larg

    @pl.when(j > 0)
    def _():
        better = lmin < min_s[...]
        min_s[...] = _carry_round(jnp.where(better, lmin, min_s[...]), j)
        idx_s[...] = jnp.where(better, larg, idx_s[...])

    @pl.when(j == _NJ - 1)
    def _():
        idx_ref[...] = idx_s[...]
        part = jnp.sum(min_s[...]).reshape(1, 1)
        prev = jnp.where(i == 0, jnp.zeros((1, 1), jnp.float32), loss_ref[...])
        tot = prev + part
        scale = _COMMIT / float(_ROWS * _CODE_DIM)
        loss_ref[...] = jnp.where(i == _NI - 1, tot * scale, tot)


def _dist_argmin(flat, emb):
    return pl.pallas_call(
        _dist_argmin_body,
        grid=(_NI, _NJ),
        in_specs=[
            pl.BlockSpec((_BR, _CODE_DIM), lambda i, j: (i, 0)),
            pl.BlockSpec((_BC, _CODE_DIM), lambda i, j: (j, 0)),
        ],
        out_specs=[
            pl.BlockSpec((_BR, 1), lambda i, j: (i, 0)),
            pl.BlockSpec((1, 1), lambda i, j: (0, 0)),
        ],
        out_shape=[
            jax.ShapeDtypeStruct((_ROWS, 1), jnp.int32),
            jax.ShapeDtypeStruct((1, 1), jnp.float32),
        ],
        scratch_shapes=[
            pltpu.VMEM((_BR, 1), jnp.float32),
            pltpu.VMEM((_BR, 1), jnp.int32),
        ],
        compiler_params=pltpu.CompilerParams(
            dimension_semantics=("arbitrary", "arbitrary"),
        ),
    )(flat, emb)


def _sc_gather(emb, idx2d):
    mesh = plsc.VectorSubcoreMesh(core_axis_name="c", subcore_axis_name="s")

    @functools.partial(
        pl.kernel,
        out_type=jax.ShapeDtypeStruct((_NW, _IDX_ROWS, 128, _CODE_DIM),
                                      jnp.float32),
        mesh=mesh,
        scratch_types=[
            pltpu.VMEM((_IDX_ROWS, 128), jnp.int32),
            pltpu.VMEM((_IDX_ROWS, 128, _CODE_DIM), jnp.float32),
            pltpu.SemaphoreType.DMA,
        ],
    )
    def gather_kernel(emb_hbm, idx_hbm, out_hbm, idx_v, rows_v, sem):
        c = lax.axis_index("c")
        s = lax.axis_index("s")
        wid = s * 2 + c
        pltpu.sync_copy(idx_hbm.at[pl.ds(wid * _IDX_ROWS, _IDX_ROWS)], idx_v)
        copies = [
            pltpu.async_copy(emb_hbm.at[idx_v.at[r]], rows_v.at[r], sem)
            for r in range(_IDX_ROWS)
        ]
        for cp in copies:
            cp.wait()
        pltpu.sync_copy(rows_v, out_hbm.at[wid])

    out = gather_kernel(emb, idx2d)
    return out.reshape(_ROWS, _CODE_DIM)


def kernel(z, embedding):
    orig_shape = z.shape
    flat = z.reshape(-1, _CODE_DIM)
    idx2d, loss11 = _dist_argmin(flat, embedding)
    idx = idx2d[:, 0]
    zq = _sc_gather(embedding, idx.reshape(_ROWS // 128, 128))
    z_q_flat = zq.reshape(orig_shape)
    loss = loss11[0, 0]
    idx_out = idx.reshape(orig_shape[0], orig_shape[1], 1)
    return (z_q_flat, loss, idx_out)


# speedup vs baseline: 1.2568x; 1.2568x over previous
import functools

import jax
import jax.numpy as jnp
from jax import lax
from jax.experimental import pallas as pl
from jax.experimental.pallas import tpu as pltpu
from jax.experimental.pallas import tpu_sc as plsc

_NUM_CODES = 8192
_CODE_DIM = 256
_COMMIT = 0.25

_ROWS = 8192
_BR = 512
_BC = 2048
_NI = _ROWS // _BR
_NJ = _NUM_CODES // _BC

_NW = 32
_RPW = _ROWS // _NW
_IDX_ROWS = _RPW // 128


def _dist_argmin_body(z_ref, e_ref, idx_ref, loss_ref, min_s, idx_s):
    i = pl.program_id(0)
    j = pl.program_id(1)
    zb = z_ref[...]
    eb = e_ref[...]
    mm2 = lax.dot_general((-2.0 * zb).astype(jnp.bfloat16),
                          eb.astype(jnp.bfloat16),
                          (((1,), (1,)), ((), ())),
                          preferred_element_type=jnp.float32)
    zn = jnp.sum(zb * zb, axis=1, keepdims=True)
    en = jnp.sum(eb * eb, axis=1)[None, :]
    dist = (zn + mm2) + en
    lmin = jnp.min(dist, axis=1, keepdims=True)
    iota = lax.broadcasted_iota(jnp.int32, dist.shape, 1)
    masked = jnp.where(dist == lmin, iota, jnp.int32(2**31 - 1))
    larg = jnp.min(masked, axis=1, keepdims=True) + j * _BC

    def _carry_round(x, jj):
        cond = jj < _NJ - 1
        return jnp.where(cond, x.astype(jnp.bfloat16).astype(jnp.float32), x)

    @pl.when(j == 0)
    def _():
        min_s[...] = _carry_round(lmin, j)
        idx_s[...] = larg

    @pl.when(j > 0)
    def _():
        better = lmin < min_s[...]
        min_s[...] = _carry_round(jnp.where(better, lmin, min_s[...]), j)
        idx_s[...] = jnp.where(better, larg, idx_s[...])

    @pl.when(j == _NJ - 1)
    def _():
        idx_ref[...] = idx_s[...]
        part = jnp.sum(min_s[...]).reshape(1, 1)
        prev = jnp.where(i == 0, jnp.zeros((1, 1), jnp.float32), loss_ref[...])
        tot = prev + part
        scale = _COMMIT / float(_ROWS * _CODE_DIM)
        loss_ref[...] = jnp.where(i == _NI - 1, tot * scale, tot)


def _dist_argmin(flat, emb):
    return pl.pallas_call(
        _dist_argmin_body,
        grid=(_NI, _NJ),
        in_specs=[
            pl.BlockSpec((_BR, _CODE_DIM), lambda i, j: (i, 0)),
            pl.BlockSpec((_BC, _CODE_DIM), lambda i, j: (j, 0)),
        ],
        out_specs=[
            pl.BlockSpec((_BR, 1), lambda i, j: (i, 0)),
            pl.BlockSpec((1, 1), lambda i, j: (0, 0)),
        ],
        out_shape=[
            jax.ShapeDtypeStruct((_ROWS, 1), jnp.int32),
            jax.ShapeDtypeStruct((1, 1), jnp.float32),
        ],
        scratch_shapes=[
            pltpu.VMEM((_BR, 1), jnp.float32),
            pltpu.VMEM((_BR, 1), jnp.int32),
        ],
        compiler_params=pltpu.CompilerParams(
            dimension_semantics=("arbitrary", "arbitrary"),
        ),
    )(flat, emb)


def _sc_gather(emb, idx2d):
    mesh = plsc.VectorSubcoreMesh(core_axis_name="c", subcore_axis_name="s")

    @functools.partial(
        pl.kernel,
        out_type=jax.ShapeDtypeStruct((_NW, _IDX_ROWS, 128, _CODE_DIM),
                                      jnp.float32),
        mesh=mesh,
        scratch_types=[
            pltpu.VMEM((_IDX_ROWS, 128), jnp.int32),
            pltpu.VMEM((_IDX_ROWS, 128, _CODE_DIM), jnp.float32),
            pltpu.SemaphoreType.DMA,
        ],
    )
    def gather_kernel(emb_hbm, idx_hbm, out_hbm, idx_v, rows_v, sem):
        c = lax.axis_index("c")
        s = lax.axis_index("s")
        wid = s * 2 + c
        pltpu.sync_copy(idx_hbm.at[pl.ds(wid * _IDX_ROWS, _IDX_ROWS)], idx_v)
        copies = [
            pltpu.async_copy(emb_hbm.at[idx_v.at[r]], rows_v.at[r], sem)
            for r in range(_IDX_ROWS)
        ]
        for cp in copies:
            cp.wait()
        pltpu.sync_copy(rows_v, out_hbm.at[wid])

    out = gather_kernel(emb, idx2d)
    return out.reshape(_ROWS, _CODE_DIM)


def kernel(z, embedding):
    orig_shape = z.shape
    flat = z.reshape(-1, _CODE_DIM)
    idx2d, loss11 = _dist_argmin(flat, embedding)
    idx = idx2d[:, 0]
    zq = _sc_gather(embedding, idx.reshape(_ROWS // 128, 128))
    z_q_flat = zq.reshape(orig_shape)
    loss = loss11[0, 0]
    idx_out = idx.reshape(orig_shape[0], orig_shape[1], 1)
    return (z_q_flat, loss, idx_out)


# speedup vs baseline: 1.4054x; 1.1183x over previous
import functools

import jax
import jax.numpy as jnp
from jax import lax
from jax.experimental import pallas as pl
from jax.experimental.pallas import tpu as pltpu
from jax.experimental.pallas import tpu_sc as plsc

_NUM_CODES = 8192
_CODE_DIM = 256
_COMMIT = 0.25

_ROWS = 8192
_BR = 1024
_BC = 2048
_NI = _ROWS // _BR
_NJ = _NUM_CODES // _BC

_NW = 32
_RPW = _ROWS // _NW
_IDX_ROWS = _RPW // 128


def _dist_argmin_body(z_ref, e_ref, idx_ref, loss_ref, min_s, idx_s):
    i = pl.program_id(0)
    j = pl.program_id(1)
    zb = z_ref[...]
    eb = e_ref[...]
    mm2 = lax.dot_general((-2.0 * zb).astype(jnp.bfloat16),
                          eb.astype(jnp.bfloat16),
                          (((1,), (1,)), ((), ())),
                          preferred_element_type=jnp.float32)
    zn = jnp.sum(zb * zb, axis=1, keepdims=True)
    en = jnp.sum(eb * eb, axis=1)[None, :]
    dist = (zn + mm2) + en
    lmin = jnp.min(dist, axis=1, keepdims=True)
    iota = lax.broadcasted_iota(jnp.int32, dist.shape, 1)
    masked = jnp.where(dist == lmin, iota, jnp.int32(2**31 - 1))
    larg = jnp.min(masked, axis=1, keepdims=True) + j * _BC

    def _carry_round(x, jj):
        cond = jj < _NJ - 1
        return jnp.where(cond, x.astype(jnp.bfloat16).astype(jnp.float32), x)

    @pl.when(j == 0)
    def _():
        min_s[...] = _carry_round(lmin, j)
        idx_s[...] = larg

    @pl.when(j > 0)
    def _():
        better = lmin < min_s[...]
        min_s[...] = _carry_round(jnp.where(better, lmin, min_s[...]), j)
        idx_s[...] = jnp.where(better, larg, idx_s[...])

    @pl.when(j == _NJ - 1)
    def _():
        idx_ref[...] = idx_s[...]
        part = jnp.sum(min_s[...]).reshape(1, 1)
        prev = jnp.where(i == 0, jnp.zeros((1, 1), jnp.float32), loss_ref[...])
        tot = prev + part
        scale = _COMMIT / float(_ROWS * _CODE_DIM)
        loss_ref[...] = jnp.where(i == _NI - 1, tot * scale, tot)


def _dist_argmin(flat, emb):
    return pl.pallas_call(
        _dist_argmin_body,
        grid=(_NI, _NJ),
        in_specs=[
            pl.BlockSpec((_BR, _CODE_DIM), lambda i, j: (i, 0)),
            pl.BlockSpec((_BC, _CODE_DIM), lambda i, j: (j, 0)),
        ],
        out_specs=[
            pl.BlockSpec((_BR, 1), lambda i, j: (i, 0)),
            pl.BlockSpec((1, 1), lambda i, j: (0, 0)),
        ],
        out_shape=[
            jax.ShapeDtypeStruct((_ROWS, 1), jnp.int32),
            jax.ShapeDtypeStruct((1, 1), jnp.float32),
        ],
        scratch_shapes=[
            pltpu.VMEM((_BR, 1), jnp.float32),
            pltpu.VMEM((_BR, 1), jnp.int32),
        ],
        compiler_params=pltpu.CompilerParams(
            dimension_semantics=("arbitrary", "arbitrary"),
        ),
    )(flat, emb)


def _sc_gather(emb, idx2d):
    mesh = plsc.VectorSubcoreMesh(core_axis_name="c", subcore_axis_name="s")

    @functools.partial(
        pl.kernel,
        out_type=jax.ShapeDtypeStruct((_NW, _IDX_ROWS, 128, _CODE_DIM),
                                      jnp.float32),
        mesh=mesh,
        scratch_types=[
            pltpu.VMEM((_IDX_ROWS, 128), jnp.int32),
            pltpu.VMEM((_IDX_ROWS, 128, _CODE_DIM), jnp.float32),
            pltpu.SemaphoreType.DMA,
        ],
    )
    def gather_kernel(emb_hbm, idx_hbm, out_hbm, idx_v, rows_v, sem):
        c = lax.axis_index("c")
        s = lax.axis_index("s")
        wid = s * 2 + c
        pltpu.sync_copy(idx_hbm.at[pl.ds(wid * _IDX_ROWS, _IDX_ROWS)], idx_v)
        copies = [
            pltpu.async_copy(emb_hbm.at[idx_v.at[r]], rows_v.at[r], sem)
            for r in range(_IDX_ROWS)
        ]
        for cp in copies:
            cp.wait()
        pltpu.sync_copy(rows_v, out_hbm.at[wid])

    out = gather_kernel(emb, idx2d)
    return out.reshape(_ROWS, _CODE_DIM)


def kernel(z, embedding):
    orig_shape = z.shape
    flat = z.reshape(-1, _CODE_DIM)
    idx2d, loss11 = _dist_argmin(flat, embedding)
    idx = idx2d[:, 0]
    zq = _sc_gather(embedding, idx.reshape(_ROWS // 128, 128))
    z_q_flat = zq.reshape(orig_shape)
    loss = loss11[0, 0]
    idx_out = idx.reshape(orig_shape[0], orig_shape[1], 1)
    return (z_q_flat, loss, idx_out)


# speedup vs baseline: 1.4609x; 1.0395x over previous
import functools

import jax
import jax.numpy as jnp
from jax import lax
from jax.experimental import pallas as pl
from jax.experimental.pallas import tpu as pltpu
from jax.experimental.pallas import tpu_sc as plsc

_NUM_CODES = 8192
_CODE_DIM = 256
_COMMIT = 0.25

_ROWS = 8192
_BR = 2048
_BC = 2048
_NI = _ROWS // _BR
_NJ = _NUM_CODES // _BC

_NW = 32
_RPW = _ROWS // _NW
_IDX_ROWS = _RPW // 128


def _dist_argmin_body(z_ref, e_ref, idx_ref, loss_ref, min_s, idx_s):
    i = pl.program_id(0)
    j = pl.program_id(1)
    zb = z_ref[...]
    eb = e_ref[...]
    mm2 = lax.dot_general((-2.0 * zb).astype(jnp.bfloat16),
                          eb.astype(jnp.bfloat16),
                          (((1,), (1,)), ((), ())),
                          preferred_element_type=jnp.float32)
    zn = jnp.sum(zb * zb, axis=1, keepdims=True)
    en = jnp.sum(eb * eb, axis=1)[None, :]
    dist = (zn + mm2) + en
    lmin = jnp.min(dist, axis=1, keepdims=True)
    iota = lax.broadcasted_iota(jnp.int32, dist.shape, 1)
    masked = jnp.where(dist == lmin, iota, jnp.int32(2**31 - 1))
    larg = jnp.min(masked, axis=1, keepdims=True) + j * _BC

    def _carry_round(x, jj):
        cond = jj < _NJ - 1
        return jnp.where(cond, x.astype(jnp.bfloat16).astype(jnp.float32), x)

    @pl.when(j == 0)
    def _():
        min_s[...] = _carry_round(lmin, j)
        idx_s[...] = larg

    @pl.when(j > 0)
    def _():
        better = lmin < min_s[...]
        min_s[...] = _carry_round(jnp.where(better, lmin, min_s[...]), j)
        idx_s[...] = jnp.where(better, larg, idx_s[...])

    @pl.when(j == _NJ - 1)
    def _():
        idx_ref[...] = idx_s[...]
        part = jnp.sum(min_s[...]).reshape(1, 1)
        prev = jnp.where(i == 0, jnp.zeros((1, 1), jnp.float32), loss_ref[...])
        tot = prev + part
        scale = _COMMIT / float(_ROWS * _CODE_DIM)
        loss_ref[...] = jnp.where(i == _NI - 1, tot * scale, tot)


def _dist_argmin(flat, emb):
    return pl.pallas_call(
        _dist_argmin_body,
        grid=(_NI, _NJ),
        in_specs=[
            pl.BlockSpec((_BR, _CODE_DIM), lambda i, j: (i, 0)),
            pl.BlockSpec((_BC, _CODE_DIM), lambda i, j: (j, 0)),
        ],
        out_specs=[
            pl.BlockSpec((_BR, 1), lambda i, j: (i, 0)),
            pl.BlockSpec((1, 1), lambda i, j: (0, 0)),
        ],
        out_shape=[
            jax.ShapeDtypeStruct((_ROWS, 1), jnp.int32),
            jax.ShapeDtypeStruct((1, 1), jnp.float32),
        ],
        scratch_shapes=[
            pltpu.VMEM((_BR, 1), jnp.float32),
            pltpu.VMEM((_BR, 1), jnp.int32),
        ],
        compiler_params=pltpu.CompilerParams(
            dimension_semantics=("arbitrary", "arbitrary"),
        ),
    )(flat, emb)


def _sc_gather(emb, idx2d):
    mesh = plsc.VectorSubcoreMesh(core_axis_name="c", subcore_axis_name="s")

    @functools.partial(
        pl.kernel,
        out_type=jax.ShapeDtypeStruct((_NW, _IDX_ROWS, 128, _CODE_DIM),
                                      jnp.float32),
        mesh=mesh,
        scratch_types=[
            pltpu.VMEM((_IDX_ROWS, 128), jnp.int32),
            pltpu.VMEM((_IDX_ROWS, 128, _CODE_DIM), jnp.float32),
            pltpu.SemaphoreType.DMA,
        ],
    )
    def gather_kernel(emb_hbm, idx_hbm, out_hbm, idx_v, rows_v, sem):
        c = lax.axis_index("c")
        s = lax.axis_index("s")
        wid = s * 2 + c
        pltpu.sync_copy(idx_hbm.at[pl.ds(wid * _IDX_ROWS, _IDX_ROWS)], idx_v)
        copies = [
            pltpu.async_copy(emb_hbm.at[idx_v.at[r]], rows_v.at[r], sem)
            for r in range(_IDX_ROWS)
        ]
        for cp in copies:
            cp.wait()
        pltpu.sync_copy(rows_v, out_hbm.at[wid])

    out = gather_kernel(emb, idx2d)
    return out.reshape(_ROWS, _CODE_DIM)


def kernel(z, embedding):
    orig_shape = z.shape
    flat = z.reshape(-1, _CODE_DIM)
    idx2d, loss11 = _dist_argmin(flat, embedding)
    idx = idx2d[:, 0]
    zq = _sc_gather(embedding, idx.reshape(_ROWS // 128, 128))
    z_q_flat = zq.reshape(orig_shape)
    loss = loss11[0, 0]
    idx_out = idx.reshape(orig_shape[0], orig_shape[1], 1)
    return (z_q_flat, loss, idx_out)
